# baseline (speedup 1.0000x reference)
import jax
import jax.numpy as jnp
from jax import lax
from jax.experimental import pallas as pl
from jax.experimental.pallas import tpu as pltpu
from jax.experimental.pallas import tpu_sc as plsc

B, C, H, W = 32, 2, 512, 512
HW = H * W
NC = 2
NS = 16
LANES = 16
IMGS_PER_CORE = B // NC
ROWS_PER_TILE = H // NS
CHUNK = ROWS_PER_TILE * W
VECS = CHUNK // LANES
SUBROWS = ROWS_PER_TILE // 2
SUBCH = SUBROWS * W
VPR = W // LANES


def _flow_body(x_hbm, out_hbm, in0_v, in1_v, vfx_v, vfy_v, cnt_v, idx_v,
               acc0_s, acc1_s, accc_s):
    c = lax.axis_index("c")
    s = lax.axis_index("s")
    base = s * CHUNK
    y0 = s * ROWS_PER_TILE

    iota_f = lax.iota(jnp.int32, LANES).astype(jnp.float32)
    fzero = jnp.zeros((LANES,), jnp.float32)
    fone = jnp.full((LANES,), 1.0, jnp.float32)
    fw = jnp.full((LANES,), float(W), jnp.float32)
    fh = jnp.full((LANES,), float(H), jnp.float32)
    mask = jnp.full((LANES,), HW - 1, jnp.int32)

    def zinit(i, carry):
        vfx_v[pl.ds(i * LANES, LANES)] = fzero
        return carry
    lax.fori_loop(0, VECS, zinit, 0)
    pltpu.sync_copy(vfx_v, acc0_s.at[pl.ds(base, CHUNK)])
    pltpu.sync_copy(vfx_v, acc1_s.at[pl.ds(base, CHUNK)])
    pltpu.sync_copy(vfx_v, accc_s.at[pl.ds(base, CHUNK)])
    plsc.subcore_barrier()

    def per_image(img, carry):
        b = c * IMGS_PER_CORE + img

        for j in range(2):
            yj = y0 + j * SUBROWS
            pj = j * SUBCH
            pltpu.sync_copy(x_hbm.at[b, 0, pl.ds(yj, SUBROWS), :], in0_v)
            pltpu.sync_copy(x_hbm.at[b, 1, pl.ds(yj, SUBROWS), :], in1_v)

            def row(r, carry1):
                yf = (yj + r).astype(jnp.float32)
                p0 = pj + r * W

                def col(q, carry2):
                    cb = q * LANES
                    p = p0 + cb
                    fx = in0_v[r, pl.ds(cb, LANES)]
                    fy = in1_v[r, pl.ds(cb, LANES)]
                    sx = iota_f + cb.astype(jnp.float32) + fx
                    sy = yf + fy
                    valid = ((sx >= fzero) & (sx < fw) &
                             (sy >= fzero) & (sy < fh))
                    tx = sx.astype(jnp.int32)
                    ty = sy.astype(jnp.int32)
                    idx_v[pl.ds(p, LANES)] = lax.bitwise_and(
                        ty * W + tx, mask)
                    vfx_v[pl.ds(p, LANES)] = jnp.where(valid, -fx, fzero)
                    vfy_v[pl.ds(p, LANES)] = jnp.where(valid, -fy, fzero)
                    cnt_v[pl.ds(p, LANES)] = jnp.where(valid, fone, fzero)
                    return carry2
                lax.fori_loop(0, VPR, col, 0)
                return carry1
            lax.fori_loop(0, SUBROWS, row, 0)

        pltpu.sync_copy(vfx_v, acc0_s.at[idx_v], add=True)
        pltpu.sync_copy(vfy_v, acc1_s.at[idx_v], add=True)
        pltpu.sync_copy(cnt_v, accc_s.at[idx_v], add=True)
        plsc.subcore_barrier()

        pltpu.sync_copy(acc0_s.at[pl.ds(base, CHUNK)], vfx_v)
        pltpu.sync_copy(acc1_s.at[pl.ds(base, CHUNK)], vfy_v)
        pltpu.sync_copy(accc_s.at[pl.ds(base, CHUNK)], cnt_v)

        for j in range(2):
            yj = y0 + j * SUBROWS
            pj = j * SUBCH

            def arow(r, carry1):
                p0 = pj + r * W

                def acol(q, carry2):
                    cb = q * LANES
                    p = p0 + cb
                    rcp = fone / jnp.maximum(cnt_v[pl.ds(p, LANES)], fone)
                    in0_v[r, pl.ds(cb, LANES)] = vfx_v[pl.ds(p, LANES)] * rcp
                    in1_v[r, pl.ds(cb, LANES)] = vfy_v[pl.ds(p, LANES)] * rcp
                    vfx_v[pl.ds(p, LANES)] = fzero
                    return carry2
                lax.fori_loop(0, VPR, acol, 0)
                return carry1
            lax.fori_loop(0, SUBROWS, arow, 0)

            pltpu.sync_copy(in0_v, out_hbm.at[b, 0, pl.ds(yj, SUBROWS), :])
            pltpu.sync_copy(in1_v, out_hbm.at[b, 1, pl.ds(yj, SUBROWS), :])

        pltpu.sync_copy(vfx_v, acc0_s.at[pl.ds(base, CHUNK)])
        pltpu.sync_copy(vfx_v, acc1_s.at[pl.ds(base, CHUNK)])
        pltpu.sync_copy(vfx_v, accc_s.at[pl.ds(base, CHUNK)])
        plsc.subcore_barrier()
        return carry

    lax.fori_loop(0, IMGS_PER_CORE, per_image, 0)


@jax.jit
def kernel(tenOne):
    mesh = plsc.VectorSubcoreMesh(
        core_axis_name="c", subcore_axis_name="s", num_cores=NC,
        num_subcores=NS)
    out = pl.kernel(
        _flow_body,
        out_type=jax.ShapeDtypeStruct((B, C, H, W), jnp.float32),
        mesh=mesh,
        scratch_types=[
            pltpu.VMEM((SUBROWS, W), jnp.float32),
            pltpu.VMEM((SUBROWS, W), jnp.float32),
            pltpu.VMEM((CHUNK,), jnp.float32),
            pltpu.VMEM((CHUNK,), jnp.float32),
            pltpu.VMEM((CHUNK,), jnp.float32),
            pltpu.VMEM((CHUNK,), jnp.int32),
            pltpu.VMEM_SHARED((HW,), jnp.float32),
            pltpu.VMEM_SHARED((HW,), jnp.float32),
            pltpu.VMEM_SHARED((HW,), jnp.float32),
        ],
    )(tenOne)
    return out


# speedup vs baseline: 1.2466x; 1.2466x over previous
import jax
import jax.numpy as jnp
from jax import lax
from jax.experimental import pallas as pl
from jax.experimental.pallas import tpu as pltpu
from jax.experimental.pallas import tpu_sc as plsc

B, C, H, W = 32, 2, 512, 512
HW = H * W
NC = 2
NS = 16
LANES = 16
IMGS_PER_CORE = B // NC
ROWS_PER_TILE = H // NS
CHUNK = ROWS_PER_TILE * W
SUBROWS = ROWS_PER_TILE // 2
SUBCH = SUBROWS * W
SUBVECS = SUBCH // LANES
VPR = W // LANES


def _flow_body(x_hbm, out_hbm, in0_v, in1_v,
               vfxA, vfyA, cntA, idxA, vfxB, vfyB, cntB, idxB,
               acc0_s, acc1_s, accc_s,
               sA0, sA1, sA2, sB0, sB1, sB2, sW0, sW1):
    c = lax.axis_index("c")
    s = lax.axis_index("s")
    base = s * CHUNK
    y0 = s * ROWS_PER_TILE

    iota_f = lax.iota(jnp.int32, LANES).astype(jnp.float32)
    fzero = jnp.zeros((LANES,), jnp.float32)
    fone = jnp.full((LANES,), 1.0, jnp.float32)
    fw = jnp.full((LANES,), float(W), jnp.float32)
    fh = jnp.full((LANES,), float(H), jnp.float32)
    mask = jnp.full((LANES,), HW - 1, jnp.int32)

    halves = ((vfxA, vfyA, cntA, idxA), (vfxB, vfyB, cntB, idxB))
    scat_sems = ((sA0, sA1, sA2), (sB0, sB1, sB2))

    def zinit(i, carry):
        vfxA[pl.ds(i * LANES, LANES)] = fzero
        return carry
    lax.fori_loop(0, SUBVECS, zinit, 0)
    for acc in (acc0_s, acc1_s, accc_s):
        pltpu.sync_copy(vfxA, acc.at[pl.ds(base, SUBCH)])
        pltpu.sync_copy(vfxA, acc.at[pl.ds(base + SUBCH, SUBCH)])
    plsc.subcore_barrier()

    def compute_half(j, vfx, vfy, cnt, idx):
        yj = y0 + j * SUBROWS

        def row(r, carry1):
            yf = (yj + r).astype(jnp.float32)
            p0 = r * W

            def col(q, carry2):
                for u in range(4):
                    cb = q * (4 * LANES) + u * LANES
                    p = p0 + cb
                    fx = in0_v[r, pl.ds(cb, LANES)]
                    fy = in1_v[r, pl.ds(cb, LANES)]
                    sx = iota_f + cb.astype(jnp.float32) + fx
                    sy = yf + fy
                    valid = ((sx >= fzero) & (sx < fw) &
                             (sy >= fzero) & (sy < fh))
                    tx = sx.astype(jnp.int32)
                    ty = sy.astype(jnp.int32)
                    idx[pl.ds(p, LANES)] = lax.bitwise_and(
                        ty * W + tx, mask)
                    vfx[pl.ds(p, LANES)] = jnp.where(valid, -fx, fzero)
                    vfy[pl.ds(p, LANES)] = jnp.where(valid, -fy, fzero)
                    cnt[pl.ds(p, LANES)] = jnp.where(valid, fone, fzero)
                return carry2
            lax.fori_loop(0, VPR // 4, col, 0)
            return carry1
        lax.fori_loop(0, SUBROWS, row, 0)

    def per_image(img, carry):
        b = c * IMGS_PER_CORE + img

        scats = []
        for j in range(2):
            vfx, vfy, cnt, idx = halves[j]
            m0, m1, m2 = scat_sems[j]
            yj = y0 + j * SUBROWS
            pltpu.sync_copy(x_hbm.at[b, 0, pl.ds(yj, SUBROWS), :], in0_v)
            pltpu.sync_copy(x_hbm.at[b, 1, pl.ds(yj, SUBROWS), :], in1_v)
            compute_half(j, vfx, vfy, cnt, idx)
            scats.append(pltpu.async_copy(vfx, acc0_s.at[idx], m0, add=True))
            scats.append(pltpu.async_copy(vfy, acc1_s.at[idx], m1, add=True))
            scats.append(pltpu.async_copy(cnt, accc_s.at[idx], m2, add=True))
        for d in scats:
            d.wait()
        plsc.subcore_barrier()

        reads = []
        for j in range(2):
            vfx, vfy, cnt, _ = halves[j]
            m0, m1, m2 = scat_sems[j]
            o = base + j * SUBCH
            reads.append(pltpu.async_copy(
                acc0_s.at[pl.ds(o, SUBCH)], vfx, m0))
            reads.append(pltpu.async_copy(
                acc1_s.at[pl.ds(o, SUBCH)], vfy, m1))
            reads.append(pltpu.async_copy(
                accc_s.at[pl.ds(o, SUBCH)], cnt, m2))
        writes = []
        zeros = []
        for j in range(2):
            vfx, vfy, cnt, _ = halves[j]
            for d in reads[3 * j:3 * j + 3]:
                d.wait()
            if j == 1:
                for d in writes:
                    d.wait()
                writes = []

            def arow(r, carry1):
                p0 = r * W

                def acol(q, carry2):
                    for u in range(4):
                        cb = q * (4 * LANES) + u * LANES
                        p = p0 + cb
                        rcp = fone / jnp.maximum(
                            cnt[pl.ds(p, LANES)], fone)
                        in0_v[r, pl.ds(cb, LANES)] = (
                            vfx[pl.ds(p, LANES)] * rcp)
                        in1_v[r, pl.ds(cb, LANES)] = (
                            vfy[pl.ds(p, LANES)] * rcp)
                        if j == 0:
                            vfx[pl.ds(p, LANES)] = fzero
                    return carry2
                lax.fori_loop(0, VPR // 4, acol, 0)
                return carry1
            lax.fori_loop(0, SUBROWS, arow, 0)

            yj = y0 + j * SUBROWS
            writes.append(pltpu.async_copy(
                in0_v, out_hbm.at[b, 0, pl.ds(yj, SUBROWS), :], sW0))
            writes.append(pltpu.async_copy(
                in1_v, out_hbm.at[b, 1, pl.ds(yj, SUBROWS), :], sW1))
            o = base + j * SUBCH
            m0, m1, m2 = scat_sems[j]
            zeros.append(pltpu.async_copy(vfxA, acc0_s.at[pl.ds(o, SUBCH)],
                                          m0))
            zeros.append(pltpu.async_copy(vfxA, acc1_s.at[pl.ds(o, SUBCH)],
                                          m1))
            zeros.append(pltpu.async_copy(vfxA, accc_s.at[pl.ds(o, SUBCH)],
                                          m2))
        for d in writes:
            d.wait()
        for d in zeros:
            d.wait()
        plsc.subcore_barrier()
        return carry

    lax.fori_loop(0, IMGS_PER_CORE, per_image, 0)


@jax.jit
def kernel(tenOne):
    mesh = plsc.VectorSubcoreMesh(
        core_axis_name="c", subcore_axis_name="s", num_cores=NC,
        num_subcores=NS)
    out = pl.kernel(
        _flow_body,
        out_type=jax.ShapeDtypeStruct((B, C, H, W), jnp.float32),
        mesh=mesh,
        scratch_types=[
            pltpu.VMEM((SUBROWS, W), jnp.float32),
            pltpu.VMEM((SUBROWS, W), jnp.float32),
            pltpu.VMEM((SUBCH,), jnp.float32),
            pltpu.VMEM((SUBCH,), jnp.float32),
            pltpu.VMEM((SUBCH,), jnp.float32),
            pltpu.VMEM((SUBCH,), jnp.int32),
            pltpu.VMEM((SUBCH,), jnp.float32),
            pltpu.VMEM((SUBCH,), jnp.float32),
            pltpu.VMEM((SUBCH,), jnp.float32),
            pltpu.VMEM((SUBCH,), jnp.int32),
            pltpu.VMEM_SHARED((HW,), jnp.float32),
            pltpu.VMEM_SHARED((HW,), jnp.float32),
            pltpu.VMEM_SHARED((HW,), jnp.float32),
            pltpu.SemaphoreType.DMA,
            pltpu.SemaphoreType.DMA,
            pltpu.SemaphoreType.DMA,
            pltpu.SemaphoreType.DMA,
            pltpu.SemaphoreType.DMA,
            pltpu.SemaphoreType.DMA,
            pltpu.SemaphoreType.DMA,
            pltpu.SemaphoreType.DMA,
        ],
    )(tenOne)
    return out
